# Initial kernel scaffold; baseline (speedup 1.0000x reference)
import jax
import jax.numpy as jnp
from jax.experimental import pallas as pl


def kernel(x, edge_indices, W, b, W_self, b_self):
    raise NotImplementedError("write your pallas kernel here")


# speedup vs baseline: 5.7619x; 5.7619x over previous
import functools

import jax
import jax.numpy as jnp
from jax import lax
from jax.experimental import pallas as pl
from jax.experimental.pallas import tpu as pltpu
from jax.experimental.pallas import tpu_sc as plsc

N_NODES = 10000
D = 128
E = 320000
R = 2
NS = 16
LANES = 16
C = 80
EPS = E // NS
NCH = EPS // C
IDXB = 10
NBLK = NCH // IDXB
WB = 80
N_PAD = 10240
ROWS_MAIN = N_PAD // NS


def _sc_segment_sum(x, src, dst):
    mesh = plsc.VectorSubcoreMesh(core_axis_name="c", subcore_axis_name="s")

    @functools.partial(
        pl.kernel,
        compiler_params=pltpu.CompilerParams(use_tc_tiling_on_sc=False),
        out_type=(
            jax.ShapeDtypeStruct((R, N_PAD, D), jnp.float32),
            jax.ShapeDtypeStruct((R, N_PAD, LANES), jnp.float32),
        ),
        mesh=mesh,
        scratch_types=[
            pltpu.VMEM_SHARED((N_PAD, D), jnp.float32),
            pltpu.VMEM_SHARED((N_PAD, LANES), jnp.float32),
            pltpu.VMEM((IDXB, C), jnp.int32),
            pltpu.VMEM((IDXB, C), jnp.int32),
            pltpu.VMEM((C, D), jnp.float32),
            pltpu.VMEM((C, LANES), jnp.float32),
            pltpu.VMEM((WB, LANES), jnp.float32),
        ],
    )
    def seg(x_hbm, src_hbm, dst_hbm, a_out, deg_out,
            a_sh, deg_sh, src_v, dst_v, rows_v, ones_v, degb_v):
        c = lax.axis_index("c")
        w = lax.axis_index("s")
        r0 = w * ROWS_MAIN

        one16 = jnp.ones((LANES,), jnp.float32)
        zero16 = jnp.zeros((LANES,), jnp.float32)

        @pl.loop(0, C)
        def _(i):
            ones_v[i] = one16
            for k in range(D // LANES):
                rows_v[i, pl.ds(k * LANES, LANES)] = zero16

        @pl.loop(0, WB)
        def _(i):
            degb_v[i] = zero16

        @pl.loop(0, ROWS_MAIN // WB)
        def _(j):
            pltpu.sync_copy(rows_v.at[pl.ds(0, WB)],
                            a_sh.at[pl.ds(r0 + j * WB, WB)])
            pltpu.sync_copy(degb_v, deg_sh.at[pl.ds(r0 + j * WB, WB)])

        plsc.subcore_barrier()

        @pl.loop(0, NBLK)
        def _(blk):
            pltpu.sync_copy(src_hbm.at[c, w, blk], src_v)
            pltpu.sync_copy(dst_hbm.at[c, w, blk], dst_v)

            @pl.loop(0, IDXB)
            def _(j):
                pltpu.sync_copy(x_hbm.at[src_v.at[j]], rows_v)
                pltpu.sync_copy(rows_v, a_sh.at[dst_v.at[j]], add=True)
                pltpu.sync_copy(ones_v, deg_sh.at[dst_v.at[j]], add=True)

        plsc.subcore_barrier()

        @pl.loop(0, ROWS_MAIN // WB)
        def _(j):
            pltpu.sync_copy(a_sh.at[pl.ds(r0 + j * WB, WB)],
                            rows_v.at[pl.ds(0, WB)])
            pltpu.sync_copy(rows_v.at[pl.ds(0, WB)],
                            a_out.at[c, pl.ds(r0 + j * WB, WB)])
            pltpu.sync_copy(deg_sh.at[pl.ds(r0 + j * WB, WB)], degb_v)
            pltpu.sync_copy(degb_v, deg_out.at[c, pl.ds(r0 + j * WB, WB)])

    return seg(x, src, dst)


def _tc_body(a_ref, deg_ref, x_ref, wt_ref, b_ref, wst_ref, bs_ref, out_ref):
    acc = jnp.dot(a_ref[0], wt_ref[0], preferred_element_type=jnp.float32,
                  precision=lax.Precision.HIGHEST)
    acc += jnp.dot(a_ref[1], wt_ref[1], preferred_element_type=jnp.float32,
                   precision=lax.Precision.HIGHEST)
    acc += jnp.dot(x_ref[...], wst_ref[...], preferred_element_type=jnp.float32,
                   precision=lax.Precision.HIGHEST)
    acc += deg_ref[0, :, 0:1] * b_ref[0]
    acc += deg_ref[1, :, 0:1] * b_ref[1]
    acc += bs_ref[...]
    out_ref[...] = acc


def _tc_combine(A, Deg, x, Wt, b2, Wst, bs2):
    BLK = 1000
    grid = (N_NODES // BLK,)
    return pl.pallas_call(
        _tc_body,
        grid=grid,
        in_specs=[
            pl.BlockSpec((R, BLK, D), lambda i: (0, i, 0)),
            pl.BlockSpec((R, BLK, LANES), lambda i: (0, i, 0)),
            pl.BlockSpec((BLK, D), lambda i: (i, 0)),
            pl.BlockSpec((R, D, D), lambda i: (0, 0, 0)),
            pl.BlockSpec((R, 1, D), lambda i: (0, 0, 0)),
            pl.BlockSpec((D, D), lambda i: (0, 0)),
            pl.BlockSpec((1, D), lambda i: (0, 0)),
        ],
        out_specs=pl.BlockSpec((BLK, D), lambda i: (i, 0)),
        out_shape=jax.ShapeDtypeStruct((N_NODES, D), jnp.float32),
    )(A, Deg, x, Wt, b2, Wst, bs2)


def kernel(x, edge_indices, W, b, W_self, b_self):
    ei = edge_indices.astype(jnp.int32)
    src = ei[:, 0, :].reshape(R, NS, NBLK, IDXB, C)
    dst = ei[:, 1, :].reshape(R, NS, NBLK, IDXB, C)
    A, Deg = _sc_segment_sum(x, src, dst)
    Wt = jnp.swapaxes(W, 1, 2)
    b2 = b.reshape(R, 1, D)
    bs2 = b_self.reshape(1, D)
    return _tc_combine(A, Deg, x, Wt, b2, Wst=jnp.transpose(W_self), bs2=bs2)


# speedup vs baseline: 8.8659x; 1.5387x over previous
import functools

import jax
import jax.numpy as jnp
from jax import lax
from jax.experimental import pallas as pl
from jax.experimental.pallas import tpu as pltpu
from jax.experimental.pallas import tpu_sc as plsc

N_NODES = 10000
D = 128
E = 320000
R = 2
NS = 16
LANES = 16
C = 80
EPS = E // NS
NCH = EPS // C
IDXB = 10
NBLK = NCH // IDXB
WB = 80
N_PAD = 10240
ROWS_MAIN = N_PAD // NS


def _sc_segment_sum(x, src, dst):
    mesh = plsc.VectorSubcoreMesh(core_axis_name="c", subcore_axis_name="s")

    @functools.partial(
        pl.kernel,
        compiler_params=pltpu.CompilerParams(use_tc_tiling_on_sc=False),
        out_type=(
            jax.ShapeDtypeStruct((R, N_PAD, D), jnp.float32),
            jax.ShapeDtypeStruct((R, N_PAD, LANES), jnp.float32),
        ),
        mesh=mesh,
        scratch_types=[
            pltpu.VMEM_SHARED((N_PAD, D), jnp.float32),
            pltpu.VMEM_SHARED((N_PAD, LANES), jnp.float32),
            pltpu.VMEM((IDXB, C), jnp.int32),
            pltpu.VMEM((IDXB, C), jnp.int32),
            pltpu.VMEM((C, D), jnp.float32),
            pltpu.VMEM((C, D), jnp.float32),
            pltpu.VMEM((C, LANES), jnp.float32),
            pltpu.VMEM((WB, LANES), jnp.float32),
            pltpu.SemaphoreType.DMA,
            pltpu.SemaphoreType.DMA,
            pltpu.SemaphoreType.DMA,
        ],
    )
    def seg(x_hbm, src_hbm, dst_hbm, a_out, deg_out,
            a_sh, deg_sh, src_v, dst_v, rows_a, rows_b, ones_v, degb_v,
            sga, sgb, sone):
        c = lax.axis_index("c")
        w = lax.axis_index("s")
        r0 = w * ROWS_MAIN

        one16 = jnp.ones((LANES,), jnp.float32)
        zero16 = jnp.zeros((LANES,), jnp.float32)

        @pl.loop(0, C)
        def _(i):
            ones_v[i] = one16
            for k in range(D // LANES):
                rows_a[i, pl.ds(k * LANES, LANES)] = zero16

        @pl.loop(0, WB)
        def _(i):
            degb_v[i] = zero16

        @pl.loop(0, ROWS_MAIN // WB)
        def _(j):
            pltpu.sync_copy(rows_a.at[pl.ds(0, WB)],
                            a_sh.at[pl.ds(r0 + j * WB, WB)])
            pltpu.sync_copy(degb_v, deg_sh.at[pl.ds(r0 + j * WB, WB)])

        plsc.subcore_barrier()

        @pl.loop(0, NBLK)
        def _(blk):
            pltpu.sync_copy(src_hbm.at[c, w, blk], src_v)
            pltpu.sync_copy(dst_hbm.at[c, w, blk], dst_v)

            ones_dmas = [
                pltpu.async_copy(ones_v, deg_sh.at[dst_v.at[j]], sone,
                                 add=True)
                for j in range(IDXB)
            ]

            ga = pltpu.async_copy(x_hbm.at[src_v.at[0]], rows_a, sga)
            for t in range(IDXB // 2):
                j0, j1 = 2 * t, 2 * t + 1
                gb = pltpu.async_copy(x_hbm.at[src_v.at[j1]], rows_b, sgb)
                ga.wait()
                pltpu.sync_copy(rows_a, a_sh.at[dst_v.at[j0]], add=True)
                if t < IDXB // 2 - 1:
                    ga = pltpu.async_copy(x_hbm.at[src_v.at[j0 + 2]],
                                          rows_a, sga)
                gb.wait()
                pltpu.sync_copy(rows_b, a_sh.at[dst_v.at[j1]], add=True)

            for dma in ones_dmas:
                dma.wait()

        plsc.subcore_barrier()

        @pl.loop(0, ROWS_MAIN // WB)
        def _(j):
            pltpu.sync_copy(a_sh.at[pl.ds(r0 + j * WB, WB)],
                            rows_a.at[pl.ds(0, WB)])
            pltpu.sync_copy(rows_a.at[pl.ds(0, WB)],
                            a_out.at[c, pl.ds(r0 + j * WB, WB)])
            pltpu.sync_copy(deg_sh.at[pl.ds(r0 + j * WB, WB)], degb_v)
            pltpu.sync_copy(degb_v, deg_out.at[c, pl.ds(r0 + j * WB, WB)])

    return seg(x, src, dst)


def _tc_body(a_ref, deg_ref, x_ref, wt_ref, b_ref, wst_ref, bs_ref, out_ref):
    acc = jnp.dot(a_ref[0], wt_ref[0], preferred_element_type=jnp.float32,
                  precision=lax.Precision.HIGHEST)
    acc += jnp.dot(a_ref[1], wt_ref[1], preferred_element_type=jnp.float32,
                   precision=lax.Precision.HIGHEST)
    acc += jnp.dot(x_ref[...], wst_ref[...], preferred_element_type=jnp.float32,
                   precision=lax.Precision.HIGHEST)
    acc += deg_ref[0, :, 0:1] * b_ref[0]
    acc += deg_ref[1, :, 0:1] * b_ref[1]
    acc += bs_ref[...]
    out_ref[...] = acc


def _tc_combine(A, Deg, x, Wt, b2, Wst, bs2):
    BLK = 1000
    grid = (N_NODES // BLK,)
    return pl.pallas_call(
        _tc_body,
        grid=grid,
        in_specs=[
            pl.BlockSpec((R, BLK, D), lambda i: (0, i, 0)),
            pl.BlockSpec((R, BLK, LANES), lambda i: (0, i, 0)),
            pl.BlockSpec((BLK, D), lambda i: (i, 0)),
            pl.BlockSpec((R, D, D), lambda i: (0, 0, 0)),
            pl.BlockSpec((R, 1, D), lambda i: (0, 0, 0)),
            pl.BlockSpec((D, D), lambda i: (0, 0)),
            pl.BlockSpec((1, D), lambda i: (0, 0)),
        ],
        out_specs=pl.BlockSpec((BLK, D), lambda i: (i, 0)),
        out_shape=jax.ShapeDtypeStruct((N_NODES, D), jnp.float32),
    )(A, Deg, x, Wt, b2, Wst, bs2)


def kernel(x, edge_indices, W, b, W_self, b_self):
    ei = edge_indices.astype(jnp.int32)
    src = ei[:, 0, :].reshape(R, NS, NBLK, IDXB, C)
    dst = ei[:, 1, :].reshape(R, NS, NBLK, IDXB, C)
    A, Deg = _sc_segment_sum(x, src, dst)
    Wt = jnp.swapaxes(W, 1, 2)
    b2 = b.reshape(R, 1, D)
    bs2 = b_self.reshape(1, D)
    return _tc_combine(A, Deg, x, Wt, b2, Wst=jnp.transpose(W_self), bs2=bs2)
